# baseline (speedup 1.0000x reference)
import jax
import jax.numpy as jnp
from jax.experimental import pallas as pl
from jax.experimental.pallas import tpu as pltpu

_EPS = 1e-5


def _bn_relu(z, g, b):
    mu = jnp.mean(z, axis=0, keepdims=True)
    var = jnp.mean((z - mu) ** 2, axis=0, keepdims=True)
    return jnp.maximum((z - mu) / jnp.sqrt(var + _EPS) * g + b, 0.0)


def _prep_body(xu, xi, Wu, bu, gu, bbu, Wi, bi, gi, bbi, hu_o, hi_o):
    zu = jnp.dot(xu[...], Wu[...], preferred_element_type=jnp.float32) + bu[...]
    hu_o[...] = _bn_relu(zu, gu[...], bbu[...])
    zi = jnp.dot(xi[...], Wi[...], preferred_element_type=jnp.float32) + bi[...]
    hi_o[...] = _bn_relu(zi, gi[...], bbi[...])


def _layer_body(agg, h, Wl, bl, Wr, g, bb, out):
    z = (jnp.dot(agg[...], Wl[...], preferred_element_type=jnp.float32)
         + bl[...]
         + jnp.dot(h[...], Wr[...], preferred_element_type=jnp.float32))
    out[...] = _bn_relu(z, g[...], bb[...]) + h[...]


def _final_body(hu, hi, W, b, ou, oi):
    ou[...] = jnp.dot(hu[...], W[...], preferred_element_type=jnp.float32) + b[...]
    oi[...] = jnp.dot(hi[...], W[...], preferred_element_type=jnp.float32) + b[...]


def _r2(v):
    return v.reshape(1, -1)


def kernel(x_user, x_item, edge_ui, edge_iu, params):
    p = params
    N_U, H = x_user.shape
    N_I = x_item.shape[0]
    f32 = jnp.float32

    prep = pl.pallas_call(
        _prep_body,
        out_shape=[jax.ShapeDtypeStruct((N_U, H), f32),
                   jax.ShapeDtypeStruct((N_I, H), f32)],
    )
    h_u, h_i = prep(x_user, x_item,
                    p['lin_user_W'], _r2(p['lin_user_b']),
                    _r2(p['in_bn_user_g']), _r2(p['in_bn_user_b']),
                    p['lin_item_W'], _r2(p['lin_item_b']),
                    _r2(p['in_bn_item_g']), _r2(p['in_bn_item_b']))

    def seg_mean(x_src, edge, n_dst):
        src, dst = edge[0], edge[1]
        msg = jnp.take(x_src, src, axis=0)
        s = jax.ops.segment_sum(msg, dst, num_segments=n_dst)
        cnt = jax.ops.segment_sum(jnp.ones((edge.shape[1], 1), f32), dst,
                                  num_segments=n_dst)
        return s / jnp.maximum(cnt, 1.0)

    layer = pl.pallas_call(
        _layer_body,
        out_shape=jax.ShapeDtypeStruct((N_U, H), f32),
    )
    for l in range(3):
        agg_i = seg_mean(h_u, edge_ui, N_I)
        agg_u = seg_mean(h_i, edge_iu, N_U)
        h_i_new = layer(agg_i, h_i, p[f'c{l}_ui_Wl'], _r2(p[f'c{l}_ui_bl']),
                        p[f'c{l}_ui_Wr'], _r2(p[f'bn{l}_i_g']), _r2(p[f'bn{l}_i_b']))
        h_u_new = layer(agg_u, h_u, p[f'c{l}_iu_Wl'], _r2(p[f'c{l}_iu_bl']),
                        p[f'c{l}_iu_Wr'], _r2(p[f'bn{l}_u_g']), _r2(p[f'bn{l}_u_b']))
        h_u, h_i = h_u_new, h_i_new

    final = pl.pallas_call(
        _final_body,
        out_shape=[jax.ShapeDtypeStruct((N_U, p['final_W'].shape[1]), f32),
                   jax.ShapeDtypeStruct((N_I, p['final_W'].shape[1]), f32)],
    )
    return final(h_u, h_i, p['final_W'], _r2(p['final_b']))


# speedup vs baseline: 6.2700x; 6.2700x over previous
import functools

import jax
import jax.numpy as jnp
from jax import lax
from jax.experimental import pallas as pl
from jax.experimental.pallas import tpu as pltpu
from jax.experimental.pallas import tpu_sc as plsc

_EPS = 1e-5

_NC = 2
_NS = 16
_NW = _NC * _NS
_C = 80
_ZR = 8
_XR = 128


def _pad_dst(n):
    q = _NS * _XR
    return ((n + q - 1) // q) * q


def _seg_sum_body(n_pad, n_chunk, h, x_hbm, src_hbm, dst_hbm, out_hbm,
                  src_v, dst_v, rows_v, zero_v, acc_sh, sem):
    cid = lax.axis_index("c")
    sid = lax.axis_index("s")
    wid = cid * _NS + sid
    rows_per_sub = n_pad // _NS

    @pl.loop(0, _ZR)
    def _(i):
        for c in range(h // 16):
            zero_v[i, pl.ds(c * 16, 16)] = jnp.zeros((16,), jnp.float32)

    @pl.loop(0, rows_per_sub // _ZR)
    def _(k):
        pltpu.sync_copy(zero_v, acc_sh.at[pl.ds(sid * rows_per_sub + k * _ZR, _ZR)])

    pltpu.sync_copy(src_hbm.at[wid], src_v)
    pltpu.sync_copy(dst_hbm.at[wid], dst_v)
    plsc.subcore_barrier()

    @pl.loop(0, n_chunk)
    def _(j):
        pltpu.async_copy(x_hbm.at[src_v.at[j]], rows_v, sem).wait()
        pltpu.sync_copy(rows_v, acc_sh.at[dst_v.at[j]], add=True)

    plsc.subcore_barrier()

    @pl.loop(0, rows_per_sub // _XR)
    def _(k):
        base = sid * rows_per_sub + k * _XR
        pltpu.sync_copy(acc_sh.at[pl.ds(base, _XR)],
                        out_hbm.at[cid].at[pl.ds(base, _XR)])


def _make_seg_sum(n_pad, n_chunk, h):
    mesh = plsc.VectorSubcoreMesh(core_axis_name="c", subcore_axis_name="s")
    return pl.kernel(
        functools.partial(_seg_sum_body, n_pad, n_chunk, h),
        out_type=jax.ShapeDtypeStruct((_NC, n_pad, h), jnp.float32),
        mesh=mesh,
        scratch_types=[
            pltpu.VMEM((n_chunk, _C), jnp.int32),
            pltpu.VMEM((n_chunk, _C), jnp.int32),
            pltpu.VMEM((_C, h), jnp.float32),
            pltpu.VMEM((_ZR, h), jnp.float32),
            pltpu.VMEM_SHARED((n_pad, h), jnp.float32),
            pltpu.SemaphoreType.DMA,
        ],
    )


def _counts_body(n_pad, n_chunk, dui_hbm, diu_hbm, cu_hbm, ci_hbm,
                 dst_v, ones_v, zero_v, acc_sh):
    cid = lax.axis_index("c")
    sid = lax.axis_index("s")
    rows_per_sub = n_pad // _NS

    @pl.loop(0, _ZR)
    def _(i):
        for c in range(8):
            zero_v[i, pl.ds(c * 16, 16)] = jnp.zeros((16,), jnp.float32)

    @pl.loop(0, _C)
    def _(i):
        for c in range(8):
            ones_v[i, pl.ds(c * 16, 16)] = jnp.ones((16,), jnp.float32)

    @pl.loop(0, rows_per_sub // _ZR)
    def _(k):
        pltpu.sync_copy(zero_v, acc_sh.at[pl.ds(sid * rows_per_sub + k * _ZR, _ZR)])

    @pl.when(cid == 0)
    def _():
        pltpu.sync_copy(diu_hbm.at[sid], dst_v)

    @pl.when(cid == 1)
    def _():
        pltpu.sync_copy(dui_hbm.at[sid], dst_v)

    plsc.subcore_barrier()

    @pl.loop(0, n_chunk)
    def _(j):
        pltpu.sync_copy(ones_v, acc_sh.at[dst_v.at[j]], add=True)

    plsc.subcore_barrier()

    @pl.loop(0, rows_per_sub // _XR)
    def _(k):
        base = sid * rows_per_sub + k * _XR

        @pl.when(cid == 0)
        def _():
            pltpu.sync_copy(acc_sh.at[pl.ds(base, _XR)],
                            cu_hbm.at[pl.ds(base, _XR)])

        @pl.when(cid == 1)
        def _():
            pltpu.sync_copy(acc_sh.at[pl.ds(base, _XR)],
                            ci_hbm.at[pl.ds(base, _XR)])


def _make_counts(n_pad, n_chunk):
    mesh = plsc.VectorSubcoreMesh(core_axis_name="c", subcore_axis_name="s")
    out = jax.ShapeDtypeStruct((n_pad, 128), jnp.float32)
    return pl.kernel(
        functools.partial(_counts_body, n_pad, n_chunk),
        out_type=[out, out],
        mesh=mesh,
        scratch_types=[
            pltpu.VMEM((n_chunk, _C), jnp.int32),
            pltpu.VMEM((_C, 128), jnp.float32),
            pltpu.VMEM((_ZR, 128), jnp.float32),
            pltpu.VMEM_SHARED((n_pad, 128), jnp.float32),
        ],
    )


def _bn_relu(z, g, b):
    mu = jnp.mean(z, axis=0, keepdims=True)
    var = jnp.mean((z - mu) ** 2, axis=0, keepdims=True)
    return jnp.maximum((z - mu) / jnp.sqrt(var + _EPS) * g + b, 0.0)


def _prep_body(xu, xi, Wu, bu, gu, bbu, Wi, bi, gi, bbi, hu_o, hi_o):
    zu = jnp.dot(xu[...], Wu[...], preferred_element_type=jnp.float32) + bu[...]
    hu_o[...] = _bn_relu(zu, gu[...], bbu[...])
    zi = jnp.dot(xi[...], Wi[...], preferred_element_type=jnp.float32) + bi[...]
    hi_o[...] = _bn_relu(zi, gi[...], bbi[...])


def _layer_body(P, cnt, h, Wl, bl, Wr, g, bb, out):
    n = h.shape[0]
    Pf = P[...]
    s = Pf[0, :n] + Pf[1, :n]
    c = cnt[...][:n, :1]
    agg = s / jnp.maximum(c, 1.0)
    z = (jnp.dot(agg, Wl[...], preferred_element_type=jnp.float32)
         + bl[...]
         + jnp.dot(h[...], Wr[...], preferred_element_type=jnp.float32))
    out[...] = _bn_relu(z, g[...], bb[...]) + h[...]


def _final_body(hu, hi, W, b, ou, oi):
    ou[...] = jnp.dot(hu[...], W[...], preferred_element_type=jnp.float32) + b[...]
    oi[...] = jnp.dot(hi[...], W[...], preferred_element_type=jnp.float32) + b[...]


def _r2(v):
    return v.reshape(1, -1)


def kernel(x_user, x_item, edge_ui, edge_iu, params):
    p = params
    N_U, H = x_user.shape
    N_I = x_item.shape[0]
    E = edge_ui.shape[1]
    f32 = jnp.float32
    per_w = E // _NW
    n_chunk = per_w // _C
    assert per_w * _NW == E and n_chunk * _C == per_w
    per_s = E // _NS
    nc_cnt = per_s // _C
    assert nc_cnt * _C == per_s

    src_ui = edge_ui[0].reshape(_NW, n_chunk, _C)
    dst_ui = edge_ui[1].reshape(_NW, n_chunk, _C)
    src_iu = edge_iu[0].reshape(_NW, n_chunk, _C)
    dst_iu = edge_iu[1].reshape(_NW, n_chunk, _C)
    dst_ui_c = edge_ui[1].reshape(_NS, nc_cnt, _C)
    dst_iu_c = edge_iu[1].reshape(_NS, nc_cnt, _C)

    prep = pl.pallas_call(
        _prep_body,
        out_shape=[jax.ShapeDtypeStruct((N_U, H), f32),
                   jax.ShapeDtypeStruct((N_I, H), f32)],
    )
    h_u, h_i = prep(x_user, x_item,
                    p['lin_user_W'], _r2(p['lin_user_b']),
                    _r2(p['in_bn_user_g']), _r2(p['in_bn_user_b']),
                    p['lin_item_W'], _r2(p['lin_item_b']),
                    _r2(p['in_bn_item_g']), _r2(p['in_bn_item_b']))

    n_pad = _pad_dst(max(N_U, N_I))
    cnt_u, cnt_i = _make_counts(n_pad, nc_cnt)(dst_ui_c, dst_iu_c)

    seg = _make_seg_sum(n_pad, n_chunk, H)

    layer = pl.pallas_call(
        _layer_body,
        out_shape=jax.ShapeDtypeStruct((N_U, H), f32),
    )
    for l in range(3):
        P_i = seg(h_u, src_ui, dst_ui)
        P_u = seg(h_i, src_iu, dst_iu)
        h_i_new = layer(P_i, cnt_i, h_i,
                        p[f'c{l}_ui_Wl'], _r2(p[f'c{l}_ui_bl']),
                        p[f'c{l}_ui_Wr'], _r2(p[f'bn{l}_i_g']), _r2(p[f'bn{l}_i_b']))
        h_u_new = layer(P_u, cnt_u, h_u,
                        p[f'c{l}_iu_Wl'], _r2(p[f'c{l}_iu_bl']),
                        p[f'c{l}_iu_Wr'], _r2(p[f'bn{l}_u_g']), _r2(p[f'bn{l}_u_b']))
        h_u, h_i = h_u_new, h_i_new

    final = pl.pallas_call(
        _final_body,
        out_shape=[jax.ShapeDtypeStruct((N_U, p['final_W'].shape[1]), f32),
                   jax.ShapeDtypeStruct((N_I, p['final_W'].shape[1]), f32)],
    )
    return final(h_u, h_i, p['final_W'], _r2(p['final_b']))
